# baseline (speedup 1.0000x reference)
import functools

import jax
import jax.numpy as jnp
from jax import lax
from jax.experimental import pallas as pl
from jax.experimental.pallas import tpu as pltpu
from jax.experimental.pallas import tpu_sc as plsc

N_NODES = 10000
DIM = 128
HID = 16
N_CLASSES = 40
NEG_SLOPE = 0.2

NC, NS, LANES = 2, 16, 16
NWORKERS = NC * NS
NPAD = 10112
ROWS_PER_TILE = NPAD // NS
PAD_DST = N_NODES + 8

N_EDGES = 320000
E2 = N_EDGES + N_NODES
CHUNK = 128
NCH = (E2 + NWORKERS * CHUNK - 1) // (NWORKERS * CHUNK)
EPT = NCH * CHUNK
EPAD = EPT * NWORKERS

_SC_PARAMS = dict(
    compiler_params=pltpu.CompilerParams(needs_layout_passes=False,
                                         use_tc_tiling_on_sc=False))

_EDGE_SCRATCH = [
    pltpu.VMEM((NCH, CHUNK), jnp.int32),
    pltpu.VMEM((NCH, CHUNK), jnp.int32),
    pltpu.VMEM((NPAD,), jnp.float32),
    pltpu.VMEM((NPAD,), jnp.float32),
    pltpu.VMEM((EPT + LANES,), jnp.float32),
    pltpu.VMEM((2, CHUNK, HID), jnp.float32),
    pltpu.VMEM((3, CHUNK, HID), jnp.float32),
    pltpu.VMEM_SHARED((NPAD, HID), jnp.float32),
    pltpu.VMEM_SHARED((NPAD,), jnp.float32),
    pltpu.SemaphoreType.DMA,
    pltpu.SemaphoreType.DMA,
]


_MID_SCRATCH = [
    pltpu.VMEM((ROWS_PER_TILE, HID), jnp.float32),
    pltpu.VMEM((ROWS_PER_TILE, HID), jnp.float32),
    pltpu.VMEM((ROWS_PER_TILE,), jnp.float32),
    pltpu.VMEM((ROWS_PER_TILE,), jnp.float32),
    pltpu.VMEM((ROWS_PER_TILE, HID), jnp.float32),
    pltpu.VMEM((ROWS_PER_TILE * HID,), jnp.float32),
    pltpu.VMEM((2, LANES), jnp.float32),
    pltpu.VMEM((LANES,), jnp.float32),
    pltpu.VMEM_SHARED((NPAD,), jnp.float32),
    pltpu.VMEM_SHARED((NPAD,), jnp.float32),
]


def _mesh():
  return plsc.VectorSubcoreMesh(
      core_axis_name="c", subcore_axis_name="s", num_cores=NC, num_subcores=NS)


def _edge_phase(sid, src_v, dst_v, asrc_v, adst_v, w_v, rows_v, scaled_v, acc,
                den, h_hbm, zacc_hbm, zden_hbm, gsem, ssem):
  pltpu.sync_copy(zacc_hbm, acc.at[pl.ds(sid * ROWS_PER_TILE, ROWS_PER_TILE)])
  pltpu.sync_copy(zden_hbm, den.at[pl.ds(sid * ROWS_PER_TILE, ROWS_PER_TILE)])

  def w_body(g, carry):
    for j in range(CHUNK // LANES):
      sv = src_v[g, pl.ds(j * LANES, LANES)]
      dv = dst_v[g, pl.ds(j * LANES, LANES)]
      a = plsc.load_gather(asrc_v, [sv]) + plsc.load_gather(adst_v, [dv])
      a = jnp.where(a >= 0.0, a, a * NEG_SLOPE)
      w_v[pl.ds(g * CHUNK + j * LANES, LANES)] = jnp.exp(a)
    return carry
  lax.fori_loop(0, NCH, w_body, 0)

  plsc.subcore_barrier()

  pltpu.async_copy(h_hbm.at[src_v.at[0]], rows_v.at[0], gsem)

  def chunk_body(g, carry):
    @pl.when(g < NCH - 1)
    def _():
      pltpu.async_copy(h_hbm.at[src_v.at[g + 1]], rows_v.at[(g + 1) % 2], gsem)
    pltpu.make_async_copy(h_hbm.at[src_v.at[g]], rows_v.at[g % 2], gsem).wait()

    @pl.when(g >= 3)
    def _():
      pltpu.make_async_copy(scaled_v.at[g % 3], acc.at[dst_v.at[g]],
                            ssem).wait()
      pltpu.make_async_copy(w_v.at[pl.ds(g * CHUNK, CHUNK)],
                            den.at[dst_v.at[g]], ssem).wait()

    for cb in range(CHUNK // LANES):
      wvec = w_v[pl.ds(g * CHUNK + cb * LANES, LANES)]
      for r in range(LANES):
        c = cb * LANES + r
        scaled_v[g % 3, c, pl.ds(0, HID)] = (
            rows_v[g % 2, c, pl.ds(0, HID)] * wvec[r])

    pltpu.async_copy(scaled_v.at[g % 3], acc.at[dst_v.at[g]], ssem, add=True)
    pltpu.async_copy(w_v.at[pl.ds(g * CHUNK, CHUNK)], den.at[dst_v.at[g]],
                     ssem, add=True)
    return carry
  lax.fori_loop(0, NCH, chunk_body, 0)

  for gg in (NCH - 3, NCH - 2, NCH - 1):
    pltpu.make_async_copy(scaled_v.at[gg % 3], acc.at[dst_v.at[gg]],
                          ssem).wait()
    pltpu.make_async_copy(w_v.at[pl.ds(gg * CHUNK, CHUNK)],
                          den.at[dst_v.at[gg]], ssem).wait()

  plsc.subcore_barrier()


def _make_edge_kernel1():
  @functools.partial(
      pl.kernel,
      out_type=(jax.ShapeDtypeStruct((NC, NPAD, HID), jnp.float32),
                jax.ShapeDtypeStruct((NC * NPAD,), jnp.float32)),
      mesh=_mesh(),
      scratch_types=_EDGE_SCRATCH,
      **_SC_PARAMS,
  )
  def edge_kernel(src_hbm, dst_hbm, asrc_hbm, adst_hbm, h_hbm, zacc_hbm,
                  zden_hbm, accg_hbm, deng_hbm, src_v, dst_v, asrc_v, adst_v,
                  w_v, rows_v, scaled_v, acc, den, gsem, ssem):
    cid = lax.axis_index("c")
    sid = lax.axis_index("s")
    wid = sid * NC + cid

    pltpu.sync_copy(src_hbm.at[wid], src_v)
    pltpu.sync_copy(dst_hbm.at[wid], dst_v)
    pltpu.sync_copy(asrc_hbm, asrc_v)
    pltpu.sync_copy(adst_hbm, adst_v)

    _edge_phase(sid, src_v, dst_v, asrc_v, adst_v, w_v, rows_v, scaled_v, acc,
                den, h_hbm, zacc_hbm, zden_hbm, gsem, ssem)

    sl = pl.ds(sid * ROWS_PER_TILE, ROWS_PER_TILE)
    pltpu.sync_copy(acc.at[sl], accg_hbm.at[cid, sl])
    pltpu.sync_copy(den.at[sl],
                    deng_hbm.at[pl.ds(cid * NPAD + sid * ROWS_PER_TILE,
                                      ROWS_PER_TILE)])

  return edge_kernel


def _make_edge_kernel2():
  @functools.partial(
      pl.kernel,
      out_type=(jax.ShapeDtypeStruct((NC, NPAD, HID), jnp.float32),
                jax.ShapeDtypeStruct((NC * NPAD,), jnp.float32),
                jax.ShapeDtypeStruct((NPAD, HID), jnp.float32)),
      mesh=_mesh(),
      scratch_types=_EDGE_SCRATCH + _MID_SCRATCH,
      **_SC_PARAMS,
  )
  def edge_kernel2(src_hbm, dst_hbm, acc1_hbm, den1_hbm, b1_hbm, att2f_hbm,
                   zacc_hbm, zden_hbm, accg_hbm, deng_hbm, g_hbm, src_v,
                   dst_v, asrc_v, adst_v, w_v, rows_v, scaled_v, acc, den,
                   gsem, ssem, t0_v, t1_v, d0_v, d1_v, g_v, gflat_v, att_v,
                   b_v, a2s_sp, a2d_sp):
    cid = lax.axis_index("c")
    sid = lax.axis_index("s")
    wid = sid * NC + cid

    pltpu.sync_copy(src_hbm.at[wid], src_v)
    pltpu.sync_copy(dst_hbm.at[wid], dst_v)

    sl = pl.ds(sid * ROWS_PER_TILE, ROWS_PER_TILE)
    pltpu.sync_copy(acc1_hbm.at[0, sl], t0_v)
    pltpu.sync_copy(acc1_hbm.at[1, sl], t1_v)
    pltpu.sync_copy(
        den1_hbm.at[pl.ds(sid * ROWS_PER_TILE, ROWS_PER_TILE)], d0_v)
    pltpu.sync_copy(
        den1_hbm.at[pl.ds(NPAD + sid * ROWS_PER_TILE, ROWS_PER_TILE)], d1_v)
    pltpu.sync_copy(b1_hbm, b_v)
    pltpu.sync_copy(att2f_hbm, att_v)

    bias = b_v[pl.ds(0, LANES)]
    att_s = att_v[0, pl.ds(0, LANES)]
    att_d = att_v[1, pl.ds(0, LANES)]
    lane = lax.iota(jnp.int32, LANES)
    n_groups = (ROWS_PER_TILE + LANES - 1) // LANES

    def mid_body(rb, carry):
      rbase = jnp.minimum(rb * LANES, ROWS_PER_TILE - LANES)
      dn = d0_v[pl.ds(rbase, LANES)] + d1_v[pl.ds(rbase, LANES)]
      dn = jnp.where(dn == 0.0, 1.0, dn)
      rcp = 1.0 / dn
      for r in range(LANES):
        row = rbase + r
        srow = t0_v[row, pl.ds(0, HID)] + t1_v[row, pl.ds(0, HID)]
        grow = jnp.maximum(srow * rcp[r] + bias, 0.0)
        g_v[row, pl.ds(0, HID)] = grow
        gflat_v[pl.ds(row * HID, HID)] = grow
      flat16 = (lane + rbase) * HID
      a2s = jnp.zeros((LANES,), jnp.float32)
      a2d = jnp.zeros((LANES,), jnp.float32)
      for j in range(HID):
        col = plsc.load_gather(gflat_v, [flat16 + j])
        a2s = a2s + col * att_s[j]
        a2d = a2d + col * att_d[j]
      asrc_v[pl.ds(sid * ROWS_PER_TILE + rbase, LANES)] = a2s
      adst_v[pl.ds(sid * ROWS_PER_TILE + rbase, LANES)] = a2d
      return carry
    lax.fori_loop(0, n_groups, mid_body, 0)

    pltpu.sync_copy(g_v, g_hbm.at[sl])
    pltpu.sync_copy(asrc_v.at[sl], a2s_sp.at[sl])
    pltpu.sync_copy(adst_v.at[sl], a2d_sp.at[sl])
    plsc.subcore_barrier()
    pltpu.sync_copy(a2s_sp, asrc_v)
    pltpu.sync_copy(a2d_sp, adst_v)

    _edge_phase(sid, src_v, dst_v, asrc_v, adst_v, w_v, rows_v, scaled_v, acc,
                den, g_hbm, zacc_hbm, zden_hbm, gsem, ssem)

    pltpu.sync_copy(acc.at[sl], accg_hbm.at[cid, sl])
    pltpu.sync_copy(den.at[sl],
                    deng_hbm.at[pl.ds(cid * NPAD + sid * ROWS_PER_TILE,
                                      ROWS_PER_TILE)])

  return edge_kernel2


@functools.lru_cache(maxsize=None)
def _sc_kernels():
  return _make_edge_kernel1(), _make_edge_kernel2()


def _tc1(x_ref, w1_ref, att_ref, att2t_ref, w2t_ref, h_ref, a_ref, att2f_ref):
  h = jnp.dot(x_ref[...], w1_ref[...], preferred_element_type=jnp.float32)
  h_ref[...] = h
  a_ref[...] = jnp.dot(h, att_ref[...], preferred_element_type=jnp.float32)
  att2f_ref[...] = jnp.dot(att2t_ref[...], w2t_ref[...],
                           preferred_element_type=jnp.float32)


def _tc3(acc_ref, den_ref, w2_ref, b_ref, o_ref):
  s = acc_ref[0] + acc_ref[1]
  den = den_ref[0] + den_ref[1]
  den = jnp.where(den == 0.0, 1.0, den)
  m = (s / den)[0:N_NODES]
  z = jnp.dot(m, w2_ref[...], preferred_element_type=jnp.float32) + b_ref[...]
  mx = jnp.max(z, axis=1, keepdims=True)
  lse = jnp.log(jnp.sum(jnp.exp(z - mx), axis=1, keepdims=True))
  o_ref[...] = z - mx - lse


def kernel(x, edge_index, W1, att_src1, att_dst1, b1, W2, att_src2, att_dst2,
           b2):
  loop = jnp.arange(N_NODES, dtype=jnp.int32)
  src = jnp.concatenate([edge_index[0].astype(jnp.int32), loop,
                         jnp.zeros((EPAD - E2,), jnp.int32)])
  dst = jnp.concatenate([edge_index[1].astype(jnp.int32), loop,
                         jnp.full((EPAD - E2,), PAD_DST, jnp.int32)])
  src = src.reshape(NWORKERS, NCH, CHUNK)
  dst = dst.reshape(NWORKERS, NCH, CHUNK)

  xp = jnp.pad(x, ((0, NPAD - N_NODES), (0, 0)))
  att1 = jnp.stack([att_src1, att_dst1], axis=1)
  att2t = jnp.stack([att_src2, att_dst2], axis=0)
  zacc = jnp.zeros((ROWS_PER_TILE, HID), jnp.float32)
  zden = jnp.zeros((ROWS_PER_TILE,), jnp.float32)
  edge1, edge2 = _sc_kernels()

  h1, a1, att2f = pl.pallas_call(
      _tc1,
      out_shape=[jax.ShapeDtypeStruct((NPAD, HID), jnp.float32),
                 jax.ShapeDtypeStruct((NPAD, 2), jnp.float32),
                 jax.ShapeDtypeStruct((2, HID), jnp.float32)],
  )(xp, W1, att1, att2t, W2.T)

  acc1, den1 = edge1(src, dst, a1[:, 0], a1[:, 1], h1, zacc, zden)

  acc2, den2, _ = edge2(src, dst, acc1, den1, b1, att2f, zacc, zden)

  out = pl.pallas_call(
      _tc3,
      out_shape=jax.ShapeDtypeStruct((N_NODES, N_CLASSES), jnp.float32),
  )(acc2, den2.reshape(NC, NPAD, 1), W2, b2.reshape(1, N_CLASSES))
  return out


# speedup vs baseline: 1.0241x; 1.0241x over previous
import functools

import jax
import jax.numpy as jnp
from jax import lax
from jax.experimental import pallas as pl
from jax.experimental.pallas import tpu as pltpu
from jax.experimental.pallas import tpu_sc as plsc

N_NODES = 10000
DIM = 128
HID = 16
N_CLASSES = 40
NEG_SLOPE = 0.2

NC, NS, LANES = 2, 16, 16
NWORKERS = NC * NS
NPAD = 10112
ROWS_PER_TILE = NPAD // NS
PAD_DST = N_NODES + 8

N_EDGES = 320000
E2 = N_EDGES + N_NODES
CHUNK = 128
NCH = (E2 + NWORKERS * CHUNK - 1) // (NWORKERS * CHUNK)
EPT = NCH * CHUNK
EPAD = EPT * NWORKERS

_SC_PARAMS = dict(
    compiler_params=pltpu.CompilerParams(needs_layout_passes=False,
                                         use_tc_tiling_on_sc=False))

_EDGE_SCRATCH = [
    pltpu.VMEM((NCH, CHUNK), jnp.int32),
    pltpu.VMEM((NCH, CHUNK), jnp.int32),
    pltpu.VMEM((NPAD,), jnp.float32),
    pltpu.VMEM((NPAD,), jnp.float32),
    pltpu.VMEM((EPT + LANES,), jnp.float32),
    pltpu.VMEM((2, CHUNK, HID), jnp.float32),
    pltpu.VMEM((3, CHUNK, HID), jnp.float32),
    pltpu.VMEM_SHARED((NPAD, HID), jnp.float32),
    pltpu.VMEM_SHARED((NPAD,), jnp.float32),
    pltpu.SemaphoreType.DMA,
    pltpu.SemaphoreType.DMA,
]


_MID_SCRATCH = [
    pltpu.VMEM((ROWS_PER_TILE, HID), jnp.float32),
    pltpu.VMEM((ROWS_PER_TILE, HID), jnp.float32),
    pltpu.VMEM((ROWS_PER_TILE,), jnp.float32),
    pltpu.VMEM((ROWS_PER_TILE,), jnp.float32),
    pltpu.VMEM((ROWS_PER_TILE, HID), jnp.float32),
    pltpu.VMEM((ROWS_PER_TILE * HID,), jnp.float32),
    pltpu.VMEM((2, LANES), jnp.float32),
    pltpu.VMEM((LANES,), jnp.float32),
    pltpu.VMEM_SHARED((NPAD,), jnp.float32),
    pltpu.VMEM_SHARED((NPAD,), jnp.float32),
]


def _mesh():
  return plsc.VectorSubcoreMesh(
      core_axis_name="c", subcore_axis_name="s", num_cores=NC, num_subcores=NS)


def _edge_phase(sid, src_v, dst_v, asrc_v, adst_v, w_v, rows_v, scaled_v, acc,
                den, h_hbm, zacc_hbm, zden_hbm, gsem, ssem):
  pltpu.sync_copy(zacc_hbm, acc.at[pl.ds(sid * ROWS_PER_TILE, ROWS_PER_TILE)])
  pltpu.sync_copy(zden_hbm, den.at[pl.ds(sid * ROWS_PER_TILE, ROWS_PER_TILE)])

  def w_body(g, carry):
    for j in range(CHUNK // LANES):
      sv = src_v[g, pl.ds(j * LANES, LANES)]
      dv = dst_v[g, pl.ds(j * LANES, LANES)]
      a = plsc.load_gather(asrc_v, [sv]) + plsc.load_gather(adst_v, [dv])
      a = jnp.where(a >= 0.0, a, a * NEG_SLOPE)
      w_v[pl.ds(g * CHUNK + j * LANES, LANES)] = jnp.exp(a)
    return carry
  lax.fori_loop(0, NCH, w_body, 0)

  plsc.subcore_barrier()

  pltpu.async_copy(h_hbm.at[src_v.at[0]], rows_v.at[0], gsem)

  def chunk_body(g, carry):
    @pl.when(g < NCH - 1)
    def _():
      pltpu.async_copy(h_hbm.at[src_v.at[g + 1]], rows_v.at[(g + 1) % 2], gsem)
    pltpu.make_async_copy(h_hbm.at[src_v.at[g]], rows_v.at[g % 2], gsem).wait()

    @pl.when(g >= 3)
    def _():
      pltpu.make_async_copy(scaled_v.at[g % 3], acc.at[dst_v.at[g]],
                            ssem).wait()
      pltpu.make_async_copy(w_v.at[pl.ds(g * CHUNK, CHUNK)],
                            den.at[dst_v.at[g]], ssem).wait()

    def row_body(c, carry2):
      wv = w_v[pl.ds(g * CHUNK + c, LANES)][0]
      scaled_v[g % 3, c, pl.ds(0, HID)] = rows_v[g % 2, c, pl.ds(0, HID)] * wv
      return carry2
    lax.fori_loop(0, CHUNK, row_body, 0)

    pltpu.async_copy(scaled_v.at[g % 3], acc.at[dst_v.at[g]], ssem, add=True)
    pltpu.async_copy(w_v.at[pl.ds(g * CHUNK, CHUNK)], den.at[dst_v.at[g]],
                     ssem, add=True)
    return carry
  lax.fori_loop(0, NCH, chunk_body, 0)

  for gg in (NCH - 3, NCH - 2, NCH - 1):
    pltpu.make_async_copy(scaled_v.at[gg % 3], acc.at[dst_v.at[gg]],
                          ssem).wait()
    pltpu.make_async_copy(w_v.at[pl.ds(gg * CHUNK, CHUNK)],
                          den.at[dst_v.at[gg]], ssem).wait()

  plsc.subcore_barrier()


def _make_edge_kernel1():
  @functools.partial(
      pl.kernel,
      out_type=(jax.ShapeDtypeStruct((NC, NPAD, HID), jnp.float32),
                jax.ShapeDtypeStruct((NC * NPAD,), jnp.float32)),
      mesh=_mesh(),
      scratch_types=_EDGE_SCRATCH,
      **_SC_PARAMS,
  )
  def edge_kernel(src_hbm, dst_hbm, asrc_hbm, adst_hbm, h_hbm, zacc_hbm,
                  zden_hbm, accg_hbm, deng_hbm, src_v, dst_v, asrc_v, adst_v,
                  w_v, rows_v, scaled_v, acc, den, gsem, ssem):
    cid = lax.axis_index("c")
    sid = lax.axis_index("s")
    wid = sid * NC + cid

    pltpu.sync_copy(src_hbm.at[wid], src_v)
    pltpu.sync_copy(dst_hbm.at[wid], dst_v)
    pltpu.sync_copy(asrc_hbm, asrc_v)
    pltpu.sync_copy(adst_hbm, adst_v)

    _edge_phase(sid, src_v, dst_v, asrc_v, adst_v, w_v, rows_v, scaled_v, acc,
                den, h_hbm, zacc_hbm, zden_hbm, gsem, ssem)

    sl = pl.ds(sid * ROWS_PER_TILE, ROWS_PER_TILE)
    pltpu.sync_copy(acc.at[sl], accg_hbm.at[cid, sl])
    pltpu.sync_copy(den.at[sl],
                    deng_hbm.at[pl.ds(cid * NPAD + sid * ROWS_PER_TILE,
                                      ROWS_PER_TILE)])

  return edge_kernel


def _make_edge_kernel2():
  @functools.partial(
      pl.kernel,
      out_type=(jax.ShapeDtypeStruct((NC, NPAD, HID), jnp.float32),
                jax.ShapeDtypeStruct((NC * NPAD,), jnp.float32),
                jax.ShapeDtypeStruct((NPAD, HID), jnp.float32)),
      mesh=_mesh(),
      scratch_types=_EDGE_SCRATCH + _MID_SCRATCH,
      **_SC_PARAMS,
  )
  def edge_kernel2(src_hbm, dst_hbm, acc1_hbm, den1_hbm, b1_hbm, att2f_hbm,
                   zacc_hbm, zden_hbm, accg_hbm, deng_hbm, g_hbm, src_v,
                   dst_v, asrc_v, adst_v, w_v, rows_v, scaled_v, acc, den,
                   gsem, ssem, t0_v, t1_v, d0_v, d1_v, g_v, gflat_v, att_v,
                   b_v, a2s_sp, a2d_sp):
    cid = lax.axis_index("c")
    sid = lax.axis_index("s")
    wid = sid * NC + cid

    pltpu.sync_copy(src_hbm.at[wid], src_v)
    pltpu.sync_copy(dst_hbm.at[wid], dst_v)

    sl = pl.ds(sid * ROWS_PER_TILE, ROWS_PER_TILE)
    pltpu.sync_copy(acc1_hbm.at[0, sl], t0_v)
    pltpu.sync_copy(acc1_hbm.at[1, sl], t1_v)
    pltpu.sync_copy(
        den1_hbm.at[pl.ds(sid * ROWS_PER_TILE, ROWS_PER_TILE)], d0_v)
    pltpu.sync_copy(
        den1_hbm.at[pl.ds(NPAD + sid * ROWS_PER_TILE, ROWS_PER_TILE)], d1_v)
    pltpu.sync_copy(b1_hbm, b_v)
    pltpu.sync_copy(att2f_hbm, att_v)

    bias = b_v[pl.ds(0, LANES)]
    att_s = att_v[0, pl.ds(0, LANES)]
    att_d = att_v[1, pl.ds(0, LANES)]
    lane = lax.iota(jnp.int32, LANES)
    n_groups = (ROWS_PER_TILE + LANES - 1) // LANES

    def mid_body(rb, carry):
      rbase = jnp.minimum(rb * LANES, ROWS_PER_TILE - LANES)
      dn = d0_v[pl.ds(rbase, LANES)] + d1_v[pl.ds(rbase, LANES)]
      dn = jnp.where(dn == 0.0, 1.0, dn)
      rcp = 1.0 / dn
      for r in range(LANES):
        row = rbase + r
        srow = t0_v[row, pl.ds(0, HID)] + t1_v[row, pl.ds(0, HID)]
        grow = jnp.maximum(srow * rcp[r] + bias, 0.0)
        g_v[row, pl.ds(0, HID)] = grow
        gflat_v[pl.ds(row * HID, HID)] = grow
      flat16 = (lane + rbase) * HID
      a2s = jnp.zeros((LANES,), jnp.float32)
      a2d = jnp.zeros((LANES,), jnp.float32)
      for j in range(HID):
        col = plsc.load_gather(gflat_v, [flat16 + j])
        a2s = a2s + col * att_s[j]
        a2d = a2d + col * att_d[j]
      asrc_v[pl.ds(sid * ROWS_PER_TILE + rbase, LANES)] = a2s
      adst_v[pl.ds(sid * ROWS_PER_TILE + rbase, LANES)] = a2d
      return carry
    lax.fori_loop(0, n_groups, mid_body, 0)

    pltpu.sync_copy(g_v, g_hbm.at[sl])
    pltpu.sync_copy(asrc_v.at[sl], a2s_sp.at[sl])
    pltpu.sync_copy(adst_v.at[sl], a2d_sp.at[sl])
    plsc.subcore_barrier()
    pltpu.sync_copy(a2s_sp, asrc_v)
    pltpu.sync_copy(a2d_sp, adst_v)

    _edge_phase(sid, src_v, dst_v, asrc_v, adst_v, w_v, rows_v, scaled_v, acc,
                den, g_hbm, zacc_hbm, zden_hbm, gsem, ssem)

    pltpu.sync_copy(acc.at[sl], accg_hbm.at[cid, sl])
    pltpu.sync_copy(den.at[sl],
                    deng_hbm.at[pl.ds(cid * NPAD + sid * ROWS_PER_TILE,
                                      ROWS_PER_TILE)])

  return edge_kernel2


@functools.lru_cache(maxsize=None)
def _sc_kernels():
  return _make_edge_kernel1(), _make_edge_kernel2()


def _tc1(x_ref, w1_ref, att_ref, att2t_ref, w2t_ref, h_ref, a_ref, att2f_ref):
  h = jnp.dot(x_ref[...], w1_ref[...], preferred_element_type=jnp.float32)
  h_ref[...] = h
  a_ref[...] = jnp.dot(h, att_ref[...], preferred_element_type=jnp.float32)
  att2f_ref[...] = jnp.dot(att2t_ref[...], w2t_ref[...],
                           preferred_element_type=jnp.float32)


def _tc3(acc_ref, den_ref, w2_ref, b_ref, o_ref):
  s = acc_ref[0] + acc_ref[1]
  den = den_ref[0] + den_ref[1]
  den = jnp.where(den == 0.0, 1.0, den)
  m = (s / den)[0:N_NODES]
  z = jnp.dot(m, w2_ref[...], preferred_element_type=jnp.float32) + b_ref[...]
  mx = jnp.max(z, axis=1, keepdims=True)
  lse = jnp.log(jnp.sum(jnp.exp(z - mx), axis=1, keepdims=True))
  o_ref[...] = z - mx - lse


def kernel(x, edge_index, W1, att_src1, att_dst1, b1, W2, att_src2, att_dst2,
           b2):
  loop = jnp.arange(N_NODES, dtype=jnp.int32)
  src = jnp.concatenate([edge_index[0].astype(jnp.int32), loop,
                         jnp.zeros((EPAD - E2,), jnp.int32)])
  dst = jnp.concatenate([edge_index[1].astype(jnp.int32), loop,
                         jnp.full((EPAD - E2,), PAD_DST, jnp.int32)])
  src = src.reshape(NWORKERS, NCH, CHUNK)
  dst = dst.reshape(NWORKERS, NCH, CHUNK)

  xp = jnp.pad(x, ((0, NPAD - N_NODES), (0, 0)))
  att1 = jnp.stack([att_src1, att_dst1], axis=1)
  att2t = jnp.stack([att_src2, att_dst2], axis=0)
  zacc = jnp.zeros((ROWS_PER_TILE, HID), jnp.float32)
  zden = jnp.zeros((ROWS_PER_TILE,), jnp.float32)
  edge1, edge2 = _sc_kernels()

  h1, a1, att2f = pl.pallas_call(
      _tc1,
      out_shape=[jax.ShapeDtypeStruct((NPAD, HID), jnp.float32),
                 jax.ShapeDtypeStruct((NPAD, 2), jnp.float32),
                 jax.ShapeDtypeStruct((2, HID), jnp.float32)],
  )(xp, W1, att1, att2t, W2.T)

  acc1, den1 = edge1(src, dst, a1[:, 0], a1[:, 1], h1, zacc, zden)

  acc2, den2, _ = edge2(src, dst, acc1, den1, b1, att2f, zacc, zden)

  out = pl.pallas_call(
      _tc3,
      out_shape=jax.ShapeDtypeStruct((N_NODES, N_CLASSES), jnp.float32),
  )(acc2, den2.reshape(NC, NPAD, 1), W2, b2.reshape(1, N_CLASSES))
  return out


# speedup vs baseline: 1.0457x; 1.0211x over previous
import functools

import jax
import jax.numpy as jnp
from jax import lax
from jax.experimental import pallas as pl
from jax.experimental.pallas import tpu as pltpu
from jax.experimental.pallas import tpu_sc as plsc

N_NODES = 10000
DIM = 128
HID = 16
N_CLASSES = 40
NEG_SLOPE = 0.2

NC, NS, LANES = 2, 16, 16
NWORKERS = NC * NS
NPAD = 10112
ROWS_PER_TILE = NPAD // NS
PAD_DST = N_NODES + 8

N_EDGES = 320000
E2 = N_EDGES + N_NODES
CHUNK = 128
NCH = (E2 + NWORKERS * CHUNK - 1) // (NWORKERS * CHUNK)
EPT = NCH * CHUNK
EPAD = EPT * NWORKERS

_SC_PARAMS = dict(
    compiler_params=pltpu.CompilerParams(needs_layout_passes=False,
                                         use_tc_tiling_on_sc=False))

_EDGE_SCRATCH = [
    pltpu.VMEM((NCH, CHUNK), jnp.int32),
    pltpu.VMEM((NCH, CHUNK), jnp.int32),
    pltpu.VMEM((NPAD,), jnp.float32),
    pltpu.VMEM((NPAD,), jnp.float32),
    pltpu.VMEM((EPT + LANES,), jnp.float32),
    pltpu.VMEM((2, CHUNK, HID), jnp.float32),
    pltpu.VMEM((3, CHUNK, HID), jnp.float32),
    pltpu.VMEM_SHARED((NPAD, HID), jnp.float32),
    pltpu.VMEM_SHARED((NPAD,), jnp.float32),
    pltpu.SemaphoreType.DMA,
    pltpu.SemaphoreType.DMA,
]


_MID_SCRATCH = [
    pltpu.VMEM((ROWS_PER_TILE, HID), jnp.float32),
    pltpu.VMEM((ROWS_PER_TILE, HID), jnp.float32),
    pltpu.VMEM((ROWS_PER_TILE,), jnp.float32),
    pltpu.VMEM((ROWS_PER_TILE,), jnp.float32),
    pltpu.VMEM((ROWS_PER_TILE, HID), jnp.float32),
    pltpu.VMEM((ROWS_PER_TILE * HID,), jnp.float32),
    pltpu.VMEM((2, LANES), jnp.float32),
    pltpu.VMEM((LANES,), jnp.float32),
    pltpu.VMEM_SHARED((NPAD,), jnp.float32),
    pltpu.VMEM_SHARED((NPAD,), jnp.float32),
]


def _mesh():
  return plsc.VectorSubcoreMesh(
      core_axis_name="c", subcore_axis_name="s", num_cores=NC, num_subcores=NS)


def _edge_phase(sid, src_v, dst_v, asrc_v, adst_v, w_v, rows_v, scaled_v, acc,
                den, h_hbm, zacc_hbm, zden_hbm, gsem, ssem):
  pltpu.sync_copy(zacc_hbm, acc.at[pl.ds(sid * ROWS_PER_TILE, ROWS_PER_TILE)])
  pltpu.sync_copy(zden_hbm, den.at[pl.ds(sid * ROWS_PER_TILE, ROWS_PER_TILE)])

  def w_body(g, carry):
    for j in range(CHUNK // LANES):
      sv = src_v[g, pl.ds(j * LANES, LANES)]
      dv = dst_v[g, pl.ds(j * LANES, LANES)]
      a = plsc.load_gather(asrc_v, [sv]) + plsc.load_gather(adst_v, [dv])
      a = jnp.where(a >= 0.0, a, a * NEG_SLOPE)
      w_v[pl.ds(g * CHUNK + j * LANES, LANES)] = jnp.exp(a)
    return carry
  lax.fori_loop(0, NCH, w_body, 0)

  plsc.subcore_barrier()

  pltpu.async_copy(h_hbm.at[src_v.at[0]], rows_v.at[0], gsem)

  def chunk_body(g, carry):
    @pl.when(g < NCH - 1)
    def _():
      pltpu.async_copy(h_hbm.at[src_v.at[g + 1]], rows_v.at[(g + 1) % 2], gsem)
    pltpu.make_async_copy(h_hbm.at[src_v.at[g]], rows_v.at[g % 2], gsem).wait()

    @pl.when(g >= 3)
    def _():
      pltpu.make_async_copy(scaled_v.at[g % 3], acc.at[dst_v.at[g]],
                            ssem).wait()
      pltpu.make_async_copy(w_v.at[pl.ds(g * CHUNK, CHUNK)],
                            den.at[dst_v.at[g]], ssem).wait()

    def row_body(c, carry2):
      wv = w_v[pl.ds(g * CHUNK + c, LANES)][0]
      scaled_v[g % 3, c, pl.ds(0, HID)] = rows_v[g % 2, c, pl.ds(0, HID)] * wv
      return carry2
    lax.fori_loop(0, CHUNK, row_body, 0)

    pltpu.async_copy(scaled_v.at[g % 3], acc.at[dst_v.at[g]], ssem, add=True)
    pltpu.async_copy(w_v.at[pl.ds(g * CHUNK, CHUNK)], den.at[dst_v.at[g]],
                     ssem, add=True)
    return carry
  lax.fori_loop(0, NCH, chunk_body, 0)

  for gg in (NCH - 3, NCH - 2, NCH - 1):
    pltpu.make_async_copy(scaled_v.at[gg % 3], acc.at[dst_v.at[gg]],
                          ssem).wait()
    pltpu.make_async_copy(w_v.at[pl.ds(gg * CHUNK, CHUNK)],
                          den.at[dst_v.at[gg]], ssem).wait()

  plsc.subcore_barrier()


def _make_fused_kernel():
  @functools.partial(
      pl.kernel,
      out_type=(jax.ShapeDtypeStruct((NC, NPAD, HID), jnp.float32),
                jax.ShapeDtypeStruct((NC * NPAD,), jnp.float32),
                jax.ShapeDtypeStruct((NC, NPAD, HID), jnp.float32),
                jax.ShapeDtypeStruct((NC * NPAD,), jnp.float32),
                jax.ShapeDtypeStruct((NPAD, HID), jnp.float32)),
      mesh=_mesh(),
      scratch_types=_EDGE_SCRATCH + _MID_SCRATCH + [
          pltpu.SemaphoreType.REGULAR,
      ],
      **_SC_PARAMS,
  )
  def fused_kernel(src_hbm, dst_hbm, asrc_hbm, adst_hbm, h_hbm, b1_hbm,
                   att2f_hbm, zacc_hbm, zden_hbm, acc1_hbm, den1_hbm,
                   accg_hbm, deng_hbm, g_hbm, src_v, dst_v, asrc_v, adst_v,
                   w_v, rows_v, scaled_v, acc, den, gsem, ssem, t0_v, t1_v,
                   d0_v, d1_v, g_v, gflat_v, att_v, b_v, a2s_sp, a2d_sp,
                   bsem):
    cid = lax.axis_index("c")
    sid = lax.axis_index("s")
    wid = sid * NC + cid
    sl = pl.ds(sid * ROWS_PER_TILE, ROWS_PER_TILE)

    pltpu.sync_copy(src_hbm.at[wid], src_v)
    pltpu.sync_copy(dst_hbm.at[wid], dst_v)
    pltpu.sync_copy(asrc_hbm, asrc_v)
    pltpu.sync_copy(adst_hbm, adst_v)

    _edge_phase(sid, src_v, dst_v, asrc_v, adst_v, w_v, rows_v, scaled_v, acc,
                den, h_hbm, zacc_hbm, zden_hbm, gsem, ssem)
    pltpu.sync_copy(acc.at[sl], acc1_hbm.at[cid, sl])
    pltpu.sync_copy(den.at[sl],
                    den1_hbm.at[pl.ds(cid * NPAD + sid * ROWS_PER_TILE,
                                      ROWS_PER_TILE)])
    plsc.subcore_barrier()
    pltpu.core_barrier(bsem, core_axis_name="c")

    pltpu.sync_copy(acc1_hbm.at[0, sl], t0_v)
    pltpu.sync_copy(acc1_hbm.at[1, sl], t1_v)
    pltpu.sync_copy(
        den1_hbm.at[pl.ds(sid * ROWS_PER_TILE, ROWS_PER_TILE)], d0_v)
    pltpu.sync_copy(
        den1_hbm.at[pl.ds(NPAD + sid * ROWS_PER_TILE, ROWS_PER_TILE)], d1_v)
    pltpu.sync_copy(b1_hbm, b_v)
    pltpu.sync_copy(att2f_hbm, att_v)

    bias = b_v[pl.ds(0, LANES)]
    att_s = att_v[0, pl.ds(0, LANES)]
    att_d = att_v[1, pl.ds(0, LANES)]
    lane = lax.iota(jnp.int32, LANES)
    n_groups = (ROWS_PER_TILE + LANES - 1) // LANES

    def mid_body(rb, carry):
      rbase = jnp.minimum(rb * LANES, ROWS_PER_TILE - LANES)
      dn = d0_v[pl.ds(rbase, LANES)] + d1_v[pl.ds(rbase, LANES)]
      dn = jnp.where(dn == 0.0, 1.0, dn)
      rcp = 1.0 / dn
      for r in range(LANES):
        row = rbase + r
        srow = t0_v[row, pl.ds(0, HID)] + t1_v[row, pl.ds(0, HID)]
        grow = jnp.maximum(srow * rcp[r] + bias, 0.0)
        g_v[row, pl.ds(0, HID)] = grow
        gflat_v[pl.ds(row * HID, HID)] = grow
      flat16 = (lane + rbase) * HID
      a2s = jnp.zeros((LANES,), jnp.float32)
      a2d = jnp.zeros((LANES,), jnp.float32)
      for j in range(HID):
        col = plsc.load_gather(gflat_v, [flat16 + j])
        a2s = a2s + col * att_s[j]
        a2d = a2d + col * att_d[j]
      asrc_v[pl.ds(sid * ROWS_PER_TILE + rbase, LANES)] = a2s
      adst_v[pl.ds(sid * ROWS_PER_TILE + rbase, LANES)] = a2d
      return carry
    lax.fori_loop(0, n_groups, mid_body, 0)

    pltpu.sync_copy(g_v, g_hbm.at[sl])
    pltpu.sync_copy(asrc_v.at[sl], a2s_sp.at[sl])
    pltpu.sync_copy(adst_v.at[sl], a2d_sp.at[sl])
    plsc.subcore_barrier()
    pltpu.sync_copy(a2s_sp, asrc_v)
    pltpu.sync_copy(a2d_sp, adst_v)

    _edge_phase(sid, src_v, dst_v, asrc_v, adst_v, w_v, rows_v, scaled_v, acc,
                den, g_hbm, zacc_hbm, zden_hbm, gsem, ssem)
    pltpu.sync_copy(acc.at[sl], accg_hbm.at[cid, sl])
    pltpu.sync_copy(den.at[sl],
                    deng_hbm.at[pl.ds(cid * NPAD + sid * ROWS_PER_TILE,
                                      ROWS_PER_TILE)])

  return fused_kernel


@functools.lru_cache(maxsize=None)
def _sc_kernels():
  return _make_fused_kernel()


def _tc1(x_ref, w1_ref, att_ref, att2t_ref, w2t_ref, h_ref, a_ref, att2f_ref):
  h = jnp.dot(x_ref[...], w1_ref[...], preferred_element_type=jnp.float32)
  h_ref[...] = h
  a_ref[...] = jnp.dot(h, att_ref[...], preferred_element_type=jnp.float32)
  att2f_ref[...] = jnp.dot(att2t_ref[...], w2t_ref[...],
                           preferred_element_type=jnp.float32)


def _tc3(acc_ref, den_ref, w2_ref, b_ref, o_ref):
  s = acc_ref[0] + acc_ref[1]
  den = den_ref[0] + den_ref[1]
  den = jnp.where(den == 0.0, 1.0, den)
  m = (s / den)[0:N_NODES]
  z = jnp.dot(m, w2_ref[...], preferred_element_type=jnp.float32) + b_ref[...]
  mx = jnp.max(z, axis=1, keepdims=True)
  lse = jnp.log(jnp.sum(jnp.exp(z - mx), axis=1, keepdims=True))
  o_ref[...] = z - mx - lse


def kernel(x, edge_index, W1, att_src1, att_dst1, b1, W2, att_src2, att_dst2,
           b2):
  loop = jnp.arange(N_NODES, dtype=jnp.int32)
  src = jnp.concatenate([edge_index[0].astype(jnp.int32), loop,
                         jnp.zeros((EPAD - E2,), jnp.int32)])
  dst = jnp.concatenate([edge_index[1].astype(jnp.int32), loop,
                         jnp.full((EPAD - E2,), PAD_DST, jnp.int32)])
  src = src.reshape(NWORKERS, NCH, CHUNK)
  dst = dst.reshape(NWORKERS, NCH, CHUNK)

  xp = jnp.pad(x, ((0, NPAD - N_NODES), (0, 0)))
  att1 = jnp.stack([att_src1, att_dst1], axis=1)
  att2t = jnp.stack([att_src2, att_dst2], axis=0)
  zacc = jnp.zeros((ROWS_PER_TILE, HID), jnp.float32)
  zden = jnp.zeros((ROWS_PER_TILE,), jnp.float32)
  edge_fused = _sc_kernels()

  h1, a1, att2f = pl.pallas_call(
      _tc1,
      out_shape=[jax.ShapeDtypeStruct((NPAD, HID), jnp.float32),
                 jax.ShapeDtypeStruct((NPAD, 2), jnp.float32),
                 jax.ShapeDtypeStruct((2, HID), jnp.float32)],
  )(xp, W1, att1, att2t, W2.T)

  _, _, acc2, den2, _ = edge_fused(src, dst, a1[:, 0], a1[:, 1], h1, b1,
                                   att2f, zacc, zden)

  out = pl.pallas_call(
      _tc3,
      out_shape=jax.ShapeDtypeStruct((N_NODES, N_CLASSES), jnp.float32),
  )(acc2, den2.reshape(NC, NPAD, 1), W2, b2.reshape(1, N_CLASSES))
  return out


# speedup vs baseline: 1.0726x; 1.0257x over previous
import functools

import jax
import jax.numpy as jnp
from jax import lax
from jax.experimental import pallas as pl
from jax.experimental.pallas import tpu as pltpu
from jax.experimental.pallas import tpu_sc as plsc

N_NODES = 10000
DIM = 128
HID = 16
N_CLASSES = 40
NEG_SLOPE = 0.2

NC, NS, LANES = 2, 16, 16
NWORKERS = NC * NS
NPAD = 10112
ROWS_PER_TILE = NPAD // NS
PAD_DST = N_NODES + 8

N_EDGES = 320000
E2 = N_EDGES + N_NODES
CHUNK = 128
NCH = (E2 + NWORKERS * CHUNK - 1) // (NWORKERS * CHUNK)
EPT = NCH * CHUNK
EPAD = EPT * NWORKERS

_SC_PARAMS = dict(
    compiler_params=pltpu.CompilerParams(needs_layout_passes=False,
                                         use_tc_tiling_on_sc=False))

_EDGE_SCRATCH = [
    pltpu.VMEM((NCH, CHUNK), jnp.int32),
    pltpu.VMEM((NCH, CHUNK), jnp.int32),
    pltpu.VMEM((2 * NPAD,), jnp.float32),
    pltpu.VMEM((EPT + LANES,), jnp.float32),
    pltpu.VMEM((2, CHUNK, HID), jnp.float32),
    pltpu.VMEM((3, CHUNK, HID), jnp.float32),
    pltpu.VMEM_SHARED((NPAD, HID), jnp.float32),
    pltpu.VMEM_SHARED((NPAD,), jnp.float32),
    pltpu.SemaphoreType.DMA,
    pltpu.SemaphoreType.DMA,
]


_MID_SCRATCH = [
    pltpu.VMEM((ROWS_PER_TILE, HID), jnp.float32),
    pltpu.VMEM((ROWS_PER_TILE, HID), jnp.float32),
    pltpu.VMEM((ROWS_PER_TILE,), jnp.float32),
    pltpu.VMEM((ROWS_PER_TILE,), jnp.float32),
    pltpu.VMEM((ROWS_PER_TILE, HID), jnp.float32),
    pltpu.VMEM((ROWS_PER_TILE * HID,), jnp.float32),
    pltpu.VMEM((2, LANES), jnp.float32),
    pltpu.VMEM((LANES,), jnp.float32),
    pltpu.VMEM_SHARED((2 * NPAD,), jnp.float32),
]


def _mesh():
  return plsc.VectorSubcoreMesh(
      core_axis_name="c", subcore_axis_name="s", num_cores=NC, num_subcores=NS)


def _edge_phase(sid, src_v, dst_v, aflat_v, w_v, rows_v, scaled_v, acc,
                den, h_hbm, zacc_hbm, zden_hbm, gsem, ssem):
  pltpu.sync_copy(zacc_hbm, acc.at[pl.ds(sid * ROWS_PER_TILE, ROWS_PER_TILE)])
  pltpu.sync_copy(zden_hbm, den.at[pl.ds(sid * ROWS_PER_TILE, ROWS_PER_TILE)])

  def w_body(g, carry):
    for j in range(CHUNK // LANES):
      sv = src_v[g, pl.ds(j * LANES, LANES)]
      dv = dst_v[g, pl.ds(j * LANES, LANES)]
      a = (plsc.load_gather(aflat_v, [sv + sv]) +
           plsc.load_gather(aflat_v, [dv + dv + 1]))
      a = jnp.where(a >= 0.0, a, a * NEG_SLOPE)
      w_v[pl.ds(g * CHUNK + j * LANES, LANES)] = jnp.exp(a)
    return carry
  lax.fori_loop(0, NCH, w_body, 0)

  plsc.subcore_barrier()

  pltpu.async_copy(h_hbm.at[src_v.at[0]], rows_v.at[0], gsem)

  def chunk_body(g, carry):
    @pl.when(g < NCH - 1)
    def _():
      pltpu.async_copy(h_hbm.at[src_v.at[g + 1]], rows_v.at[(g + 1) % 2], gsem)
    pltpu.make_async_copy(h_hbm.at[src_v.at[g]], rows_v.at[g % 2], gsem).wait()

    @pl.when(g >= 3)
    def _():
      pltpu.make_async_copy(scaled_v.at[g % 3], acc.at[dst_v.at[g]],
                            ssem).wait()
      pltpu.make_async_copy(w_v.at[pl.ds(g * CHUNK, CHUNK)],
                            den.at[dst_v.at[g]], ssem).wait()

    def row_body(c, carry2):
      wv = w_v[pl.ds(g * CHUNK + c, LANES)][0]
      scaled_v[g % 3, c, pl.ds(0, HID)] = rows_v[g % 2, c, pl.ds(0, HID)] * wv
      return carry2
    lax.fori_loop(0, CHUNK, row_body, 0)

    pltpu.async_copy(scaled_v.at[g % 3], acc.at[dst_v.at[g]], ssem, add=True)
    pltpu.async_copy(w_v.at[pl.ds(g * CHUNK, CHUNK)], den.at[dst_v.at[g]],
                     ssem, add=True)
    return carry
  lax.fori_loop(0, NCH, chunk_body, 0)

  for gg in (NCH - 3, NCH - 2, NCH - 1):
    pltpu.make_async_copy(scaled_v.at[gg % 3], acc.at[dst_v.at[gg]],
                          ssem).wait()
    pltpu.make_async_copy(w_v.at[pl.ds(gg * CHUNK, CHUNK)],
                          den.at[dst_v.at[gg]], ssem).wait()

  plsc.subcore_barrier()


def _make_fused_kernel():
  @functools.partial(
      pl.kernel,
      out_type=(jax.ShapeDtypeStruct((NC, NPAD, HID), jnp.float32),
                jax.ShapeDtypeStruct((NC * NPAD,), jnp.float32),
                jax.ShapeDtypeStruct((NC, NPAD, HID), jnp.float32),
                jax.ShapeDtypeStruct((NC * NPAD,), jnp.float32),
                jax.ShapeDtypeStruct((NPAD, HID), jnp.float32)),
      mesh=_mesh(),
      scratch_types=_EDGE_SCRATCH + _MID_SCRATCH + [
          pltpu.SemaphoreType.REGULAR,
      ],
      **_SC_PARAMS,
  )
  def fused_kernel(src_hbm, dst_hbm, aflat_hbm, h_hbm, b1_hbm,
                   att2f_hbm, zacc_hbm, zden_hbm, acc1_hbm, den1_hbm,
                   accg_hbm, deng_hbm, g_hbm, src_v, dst_v, aflat_v,
                   w_v, rows_v, scaled_v, acc, den, gsem, ssem, t0_v, t1_v,
                   d0_v, d1_v, g_v, gflat_v, att_v, b_v, a2_sp,
                   bsem):
    cid = lax.axis_index("c")
    sid = lax.axis_index("s")
    wid = sid * NC + cid
    sl = pl.ds(sid * ROWS_PER_TILE, ROWS_PER_TILE)

    pltpu.sync_copy(src_hbm.at[wid], src_v)
    pltpu.sync_copy(dst_hbm.at[wid], dst_v)
    pltpu.sync_copy(aflat_hbm, aflat_v)

    _edge_phase(sid, src_v, dst_v, aflat_v, w_v, rows_v, scaled_v, acc,
                den, h_hbm, zacc_hbm, zden_hbm, gsem, ssem)
    pltpu.sync_copy(acc.at[sl], acc1_hbm.at[cid, sl])
    pltpu.sync_copy(den.at[sl],
                    den1_hbm.at[pl.ds(cid * NPAD + sid * ROWS_PER_TILE,
                                      ROWS_PER_TILE)])
    plsc.subcore_barrier()
    pltpu.core_barrier(bsem, core_axis_name="c")

    pltpu.sync_copy(acc1_hbm.at[0, sl], t0_v)
    pltpu.sync_copy(acc1_hbm.at[1, sl], t1_v)
    pltpu.sync_copy(
        den1_hbm.at[pl.ds(sid * ROWS_PER_TILE, ROWS_PER_TILE)], d0_v)
    pltpu.sync_copy(
        den1_hbm.at[pl.ds(NPAD + sid * ROWS_PER_TILE, ROWS_PER_TILE)], d1_v)
    pltpu.sync_copy(b1_hbm, b_v)
    pltpu.sync_copy(att2f_hbm, att_v)

    bias = b_v[pl.ds(0, LANES)]
    att_s = att_v[0, pl.ds(0, LANES)]
    att_d = att_v[1, pl.ds(0, LANES)]
    lane = lax.iota(jnp.int32, LANES)
    n_groups = (ROWS_PER_TILE + LANES - 1) // LANES

    def mid_body(rb, carry):
      rbase = jnp.minimum(rb * LANES, ROWS_PER_TILE - LANES)
      dn = d0_v[pl.ds(rbase, LANES)] + d1_v[pl.ds(rbase, LANES)]
      dn = jnp.where(dn == 0.0, 1.0, dn)
      rcp = 1.0 / dn
      for r in range(LANES):
        row = rbase + r
        srow = t0_v[row, pl.ds(0, HID)] + t1_v[row, pl.ds(0, HID)]
        grow = jnp.maximum(srow * rcp[r] + bias, 0.0)
        g_v[row, pl.ds(0, HID)] = grow
        gflat_v[pl.ds(row * HID, HID)] = grow
      flat16 = (lane + rbase) * HID
      a2s = jnp.zeros((LANES,), jnp.float32)
      a2d = jnp.zeros((LANES,), jnp.float32)
      for j in range(HID):
        col = plsc.load_gather(gflat_v, [flat16 + j])
        a2s = a2s + col * att_s[j]
        a2d = a2d + col * att_d[j]
      base2 = (sid * ROWS_PER_TILE + rbase) * 2
      idx2 = lane + lane + base2
      plsc.store_scatter(aflat_v, [idx2], a2s)
      plsc.store_scatter(aflat_v, [idx2 + 1], a2d)
      return carry
    lax.fori_loop(0, n_groups, mid_body, 0)

    pltpu.sync_copy(g_v, g_hbm.at[sl])
    sl2 = pl.ds(2 * sid * ROWS_PER_TILE, 2 * ROWS_PER_TILE)
    pltpu.sync_copy(aflat_v.at[sl2], a2_sp.at[sl2])
    plsc.subcore_barrier()
    pltpu.sync_copy(a2_sp, aflat_v)

    _edge_phase(sid, src_v, dst_v, aflat_v, w_v, rows_v, scaled_v, acc,
                den, g_hbm, zacc_hbm, zden_hbm, gsem, ssem)
    pltpu.sync_copy(acc.at[sl], accg_hbm.at[cid, sl])
    pltpu.sync_copy(den.at[sl],
                    deng_hbm.at[pl.ds(cid * NPAD + sid * ROWS_PER_TILE,
                                      ROWS_PER_TILE)])

  return fused_kernel


@functools.lru_cache(maxsize=None)
def _sc_kernels():
  return _make_fused_kernel()


def _tc1(x_ref, w1_ref, att_ref, att2t_ref, w2_ref, h_ref, a_ref, att2f_ref):
  h = jnp.dot(x_ref[...], w1_ref[...], preferred_element_type=jnp.float32)
  h_ref[0:N_NODES] = h
  h_ref[N_NODES:NPAD] = jnp.zeros((NPAD - N_NODES, HID), jnp.float32)
  a = jnp.dot(h, att_ref[...], preferred_element_type=jnp.float32)
  a_ref[0:N_NODES] = a
  a_ref[N_NODES:NPAD] = jnp.zeros((NPAD - N_NODES, 2), jnp.float32)
  att2f_ref[...] = lax.dot_general(
      att2t_ref[...], w2_ref[...], (((1,), (1,)), ((), ())),
      preferred_element_type=jnp.float32)


def _tc3(acc_ref, den_ref, w2_ref, b_ref, o_ref):
  s = acc_ref[0] + acc_ref[1]
  den = den_ref[0] + den_ref[1]
  den = jnp.where(den == 0.0, 1.0, den)
  m = (s / den)[0:N_NODES]
  z = jnp.dot(m, w2_ref[...], preferred_element_type=jnp.float32) + b_ref[...]
  mx = jnp.max(z, axis=1, keepdims=True)
  lse = jnp.log(jnp.sum(jnp.exp(z - mx), axis=1, keepdims=True))
  o_ref[...] = z - mx - lse


def kernel(x, edge_index, W1, att_src1, att_dst1, b1, W2, att_src2, att_dst2,
           b2):
  loop = jnp.arange(N_NODES, dtype=jnp.int32)
  src = jnp.concatenate([edge_index[0].astype(jnp.int32), loop,
                         jnp.zeros((EPAD - E2,), jnp.int32)])
  dst = jnp.concatenate([edge_index[1].astype(jnp.int32), loop,
                         jnp.full((EPAD - E2,), PAD_DST, jnp.int32)])
  src = src.reshape(NWORKERS, NCH, CHUNK)
  dst = dst.reshape(NWORKERS, NCH, CHUNK)

  att1 = jnp.stack([att_src1, att_dst1], axis=1)
  att2t = jnp.stack([att_src2, att_dst2], axis=0)
  zacc = jnp.zeros((ROWS_PER_TILE, HID), jnp.float32)
  zden = jnp.zeros((ROWS_PER_TILE,), jnp.float32)
  edge_fused = _sc_kernels()

  h1, a1, att2f = pl.pallas_call(
      _tc1,
      out_shape=[jax.ShapeDtypeStruct((NPAD, HID), jnp.float32),
                 jax.ShapeDtypeStruct((NPAD, 2), jnp.float32),
                 jax.ShapeDtypeStruct((2, HID), jnp.float32)],
  )(x, W1, att1, att2t, W2)

  _, _, acc2, den2, _ = edge_fused(src, dst, a1.reshape(2 * NPAD), h1, b1,
                                   att2f, zacc, zden)

  out = pl.pallas_call(
      _tc3,
      out_shape=jax.ShapeDtypeStruct((N_NODES, N_CLASSES), jnp.float32),
  )(acc2, den2.reshape(NC, NPAD, 1), W2, b2.reshape(1, N_CLASSES))
  return out
